# baseline (speedup 1.0000x reference)
import functools

import jax
import jax.numpy as jnp
from jax import lax
from jax.experimental import pallas as pl
from jax.experimental.pallas import tpu as pltpu
from jax.experimental.pallas import tpu_sc as plsc

NC = 2
NS = 16
NW = NC * NS
K = 128
LANES = 16

N = 10000
NPAD = 10240
F = 128


def _make_deg_kernel(C0, C1):
    cmax = max(C0, C1)
    slc = NPAD // NS
    mesh = plsc.VectorSubcoreMesh(core_axis_name="c", subcore_axis_name="s")

    dgrp = 10

    @functools.partial(
        pl.kernel,
        out_type=jax.ShapeDtypeStruct((NC, NPAD), jnp.float32),
        mesh=mesh,
        scratch_types=[
            pltpu.VMEM((cmax, 1, K), jnp.int32),
            pltpu.VMEM((K,), jnp.float32),
            pltpu.VMEM((slc,), jnp.float32),
            pltpu.VMEM_SHARED((NPAD,), jnp.float32),
            pltpu.SemaphoreType.DMA,
        ],
    )
    def deg_k(dst_hbm, out_hbm, idx_v, ones_v, zbuf, deg_sh, sem):
        c = lax.axis_index("c")
        s = lax.axis_index("s")
        for j in range(K // LANES):
            ones_v[pl.ds(j * LANES, LANES)] = jnp.ones((LANES,), jnp.float32)
        for j in range(slc // LANES):
            zbuf[pl.ds(j * LANES, LANES)] = jnp.zeros((LANES,), jnp.float32)
        pltpu.sync_copy(zbuf, deg_sh.at[pl.ds(s * slc, slc)])
        plsc.subcore_barrier()

        def run(base, nchunks):
            pltpu.sync_copy(
                dst_hbm.at[pl.ds(pl.multiple_of(base, 2), nchunks)],
                idx_v.at[pl.ds(0, nchunks)])

            def body(grp, carry):
                ds = [
                    pltpu.async_copy(ones_v,
                                     deg_sh.at[idx_v.at[grp * dgrp + b, 0]],
                                     sem, add=True)
                    for b in range(dgrp)
                ]
                for d in ds:
                    d.wait()
                return carry

            lax.fori_loop(0, nchunks // dgrp, body, 0)

        @pl.when(c == 0)
        def _():
            run(s * C0, C0)

        if C1 > 0:
            @pl.when(c == 1)
            def _():
                run(NS * C0 + s * C1, C1)

        plsc.subcore_barrier()
        pltpu.sync_copy(deg_sh.at[pl.ds(s * slc, slc)],
                        out_hbm.at[c, pl.ds(s * slc, slc)])

    return deg_k


NBUF = 2


def _make_scatter_kernel(C0, C1):
    cmax = max(C0, C1)
    rps = NPAD // NS
    mesh = plsc.VectorSubcoreMesh(core_axis_name="c", subcore_axis_name="s")

    @functools.partial(
        pl.kernel,
        out_type=jax.ShapeDtypeStruct((NC, NPAD, F), jnp.float32),
        mesh=mesh,
        scratch_types=[
            pltpu.VMEM((NBUF, 1, K), jnp.int32),
            pltpu.VMEM((NBUF, 1, K), jnp.int32),
            pltpu.VMEM((NBUF, K, F), jnp.float32),
            pltpu.VMEM_SHARED((NPAD, F), jnp.float32),
            pltpu.SemaphoreType.DMA((NBUF,)),
            pltpu.SemaphoreType.DMA((NBUF,)),
            pltpu.SemaphoreType.DMA((NBUF,)),
            pltpu.SemaphoreType.DMA,
        ],
    )
    def scat_k(src_hbm, dst_hbm, g_hbm, out_hbm, idx_s, idx_d, rows, acc_sh,
               sem_i, sem_j, sem_g, sem_s):
        c = lax.axis_index("c")
        s = lax.axis_index("s")

        def zbody(i, carry):
            for j in range(F // LANES):
                rows[0, i, pl.ds(j * LANES, LANES)] = jnp.zeros(
                    (LANES,), jnp.float32)
            return carry

        lax.fori_loop(0, K, zbody, 0)
        for t in range(rps // K):
            pltpu.sync_copy(rows.at[0], acc_sh.at[pl.ds(s * rps + t * K, K)])
        plsc.subcore_barrier()

        def run(cbase, nchunks):
            for b in range(NBUF):
                pltpu.async_copy(src_hbm.at[cbase + b], idx_s.at[b],
                                 sem_i.at[b])
                pltpu.async_copy(dst_hbm.at[cbase + b], idx_d.at[b],
                                 sem_j.at[b])

            def body(grp, carry):
                base = grp * NBUF
                gds = []
                for b in range(NBUF):
                    pltpu.make_async_copy(src_hbm.at[cbase + base + b],
                                          idx_s.at[b], sem_i.at[b]).wait()
                    gds.append(
                        pltpu.async_copy(g_hbm.at[idx_s.at[b, 0]], rows.at[b],
                                         sem_g.at[b]))
                sds = []
                for b in range(NBUF):
                    gds[b].wait()
                    nc = base + b + NBUF

                    @pl.when(nc < nchunks)
                    def _():
                        pltpu.async_copy(src_hbm.at[cbase + nc], idx_s.at[b],
                                         sem_i.at[b])

                    pltpu.make_async_copy(dst_hbm.at[cbase + base + b],
                                          idx_d.at[b], sem_j.at[b]).wait()
                    sds.append(
                        pltpu.async_copy(rows.at[b],
                                         acc_sh.at[idx_d.at[b, 0]],
                                         sem_s, add=True))
                for d in sds:
                    d.wait()
                for b in range(NBUF):
                    nc = base + b + NBUF

                    @pl.when(nc < nchunks)
                    def _():
                        pltpu.async_copy(dst_hbm.at[cbase + nc], idx_d.at[b],
                                         sem_j.at[b])

                return carry

            lax.fori_loop(0, nchunks // NBUF, body, 0)

        @pl.when(c == 0)
        def _():
            run(s * C0, C0)

        if C1 > 0:
            @pl.when(c == 1)
            def _():
                run(NS * C0 + s * C1, C1)

        plsc.subcore_barrier()
        for t in range(rps // K):
            pltpu.sync_copy(acc_sh.at[pl.ds(s * rps + t * K, K)],
                            out_hbm.at[c, pl.ds(s * rps + t * K, K)])

    return scat_k


BR = 400


def _tc_dinv(degp):
    def body(degp_ref, dinv_ref):
        dinv_ref[...] = lax.rsqrt(degp_ref[0] + degp_ref[1] + 1.0)

    return pl.pallas_call(
        body,
        in_specs=[pl.BlockSpec((NC, NPAD), lambda: (0, 0))],
        out_specs=pl.BlockSpec((NPAD,), lambda: (0,)),
        out_shape=jax.ShapeDtypeStruct((NPAD,), jnp.float32),
    )(degp).reshape(NPAD, 1)


def _tc_prep(x, W1, dinv):
    def body(x_ref, w_ref, dinv_ref, g_ref):
        h = lax.dot_general(x_ref[...], w_ref[...], (((1,), (0,)), ((), ())),
                            preferred_element_type=jnp.float32)
        g_ref[...] = h * dinv_ref[...]

    return pl.pallas_call(
        body,
        grid=(N // BR,),
        in_specs=[
            pl.BlockSpec((BR, F), lambda i: (i, 0)),
            pl.BlockSpec((F, F), lambda i: (0, 0)),
            pl.BlockSpec((BR, 1), lambda i: (i, 0)),
        ],
        out_specs=pl.BlockSpec((BR, F), lambda i: (i, 0)),
        out_shape=jax.ShapeDtypeStruct((N, F), jnp.float32),
    )(x, W1, dinv)


def _tc_mid(p, g1, dinv, b1, W2):
    def body(p_ref, g_ref, dinv_ref, b_ref, w_ref, h_ref, g2_ref):
        acc = p_ref[0] + p_ref[1] + g_ref[...]
        dv = dinv_ref[...]
        h1 = jnp.maximum(acc * dv + b_ref[...], 0.0)
        h_ref[...] = h1
        g2 = lax.dot_general(h1, w_ref[...], (((1,), (0,)), ((), ())),
                             preferred_element_type=jnp.float32)
        g2_ref[...] = g2 * dv

    return pl.pallas_call(
        body,
        grid=(N // BR,),
        in_specs=[
            pl.BlockSpec((NC, BR, F), lambda i: (0, i, 0)),
            pl.BlockSpec((BR, F), lambda i: (i, 0)),
            pl.BlockSpec((BR, 1), lambda i: (i, 0)),
            pl.BlockSpec((1, F), lambda i: (0, 0)),
            pl.BlockSpec((F, F), lambda i: (0, 0)),
        ],
        out_specs=[
            pl.BlockSpec((BR, F), lambda i: (i, 0)),
            pl.BlockSpec((BR, F), lambda i: (i, 0)),
        ],
        out_shape=[
            jax.ShapeDtypeStruct((N, F), jnp.float32),
            jax.ShapeDtypeStruct((N, F), jnp.float32),
        ],
    )(p, g1, dinv, b1, W2)


def _tc_final(p, g2, dinv, b2, h1, linw, linb):
    ncls = linw.shape[-1]

    def body(p_ref, g_ref, dinv_ref, b_ref, h1_ref, lw_ref, lb_ref, out_ref):
        acc = p_ref[0] + p_ref[1] + g_ref[...]
        h2 = jnp.maximum(acc * dinv_ref[...] + b_ref[...], 0.0)
        logits = (
            lax.dot_general(h1_ref[...], lw_ref[0], (((1,), (0,)), ((), ())),
                            preferred_element_type=jnp.float32)
            + lax.dot_general(h2, lw_ref[1], (((1,), (0,)), ((), ())),
                              preferred_element_type=jnp.float32)
            + lb_ref[...]
        )
        m = jnp.max(logits, axis=1, keepdims=True)
        ex = jnp.exp(logits - m)
        out_ref[...] = (logits - m) - jnp.log(jnp.sum(ex, axis=1, keepdims=True))

    return pl.pallas_call(
        body,
        grid=(N // BR,),
        in_specs=[
            pl.BlockSpec((NC, BR, F), lambda i: (0, i, 0)),
            pl.BlockSpec((BR, F), lambda i: (i, 0)),
            pl.BlockSpec((BR, 1), lambda i: (i, 0)),
            pl.BlockSpec((1, F), lambda i: (0, 0)),
            pl.BlockSpec((BR, F), lambda i: (i, 0)),
            pl.BlockSpec((2, F, ncls), lambda i: (0, 0, 0)),
            pl.BlockSpec((1, ncls), lambda i: (0, 0)),
        ],
        out_specs=pl.BlockSpec((BR, ncls), lambda i: (i, 0)),
        out_shape=jax.ShapeDtypeStruct((N, ncls), jnp.float32),
    )(p, g2, dinv, b2, h1, linw, linb)


def kernel(x, edge_index, W1, b1, W2, b2, lin_W, lin_b):
    E = edge_index.shape[1]
    src = edge_index[0].astype(jnp.int32)
    dst = edge_index[1].astype(jnp.int32)
    C0, C1 = 130, 30
    assert NS * (C0 + C1) * K >= E
    epad = NS * (C0 + C1) * K
    padn = epad - E
    src = jnp.concatenate([src, jnp.zeros((padn,), jnp.int32)]).reshape(-1, 1, K)
    dst = jnp.concatenate(
        [dst, N + (jnp.arange(padn, dtype=jnp.int32) % K)]).reshape(-1, 1, K)

    degp = _make_deg_kernel(C0, C1)(dst)
    dinv = _tc_dinv(degp)
    g1 = _tc_prep(x, W1, dinv)
    scat = _make_scatter_kernel(C0, C1)
    p1 = scat(src, dst, g1)
    h1, g2 = _tc_mid(p1, g1, dinv, b1.reshape(1, F), W2)
    p2 = scat(src, dst, g2)
    out = _tc_final(p2, g2, dinv, b2.reshape(1, F), h1,
                    lin_W.reshape(2, F, -1), lin_b.reshape(1, -1))
    return out


# speedup vs baseline: 1.0052x; 1.0052x over previous
import functools

import jax
import jax.numpy as jnp
from jax import lax
from jax.experimental import pallas as pl
from jax.experimental.pallas import tpu as pltpu
from jax.experimental.pallas import tpu_sc as plsc

NC = 2
NS = 16
NW = NC * NS
K = 128
LANES = 16

N = 10000
NPAD = 10240
F = 128


def _make_deg_kernel(C0, C1):
    cmax = max(C0, C1)
    slc = NPAD // NS
    mesh = plsc.VectorSubcoreMesh(core_axis_name="c", subcore_axis_name="s")

    dgrp = 10

    @functools.partial(
        pl.kernel,
        out_type=jax.ShapeDtypeStruct((NC, NPAD), jnp.float32),
        mesh=mesh,
        scratch_types=[
            pltpu.VMEM((cmax, 1, K), jnp.int32),
            pltpu.VMEM((K,), jnp.float32),
            pltpu.VMEM((slc,), jnp.float32),
            pltpu.VMEM_SHARED((NPAD,), jnp.float32),
            pltpu.SemaphoreType.DMA,
        ],
    )
    def deg_k(dst_hbm, out_hbm, idx_v, ones_v, zbuf, deg_sh, sem):
        c = lax.axis_index("c")
        s = lax.axis_index("s")
        for j in range(K // LANES):
            ones_v[pl.ds(j * LANES, LANES)] = jnp.ones((LANES,), jnp.float32)
        for j in range(slc // LANES):
            zbuf[pl.ds(j * LANES, LANES)] = jnp.zeros((LANES,), jnp.float32)
        pltpu.sync_copy(zbuf, deg_sh.at[pl.ds(s * slc, slc)])
        plsc.subcore_barrier()

        def run(base, nchunks):
            pltpu.sync_copy(
                dst_hbm.at[pl.ds(pl.multiple_of(base, 2), nchunks)],
                idx_v.at[pl.ds(0, nchunks)])

            def body(grp, carry):
                ds = [
                    pltpu.async_copy(ones_v,
                                     deg_sh.at[idx_v.at[grp * dgrp + b, 0]],
                                     sem, add=True)
                    for b in range(dgrp)
                ]
                for d in ds:
                    d.wait()
                return carry

            lax.fori_loop(0, nchunks // dgrp, body, 0)

        @pl.when(c == 0)
        def _():
            run(s * C0, C0)

        if C1 > 0:
            @pl.when(c == 1)
            def _():
                run(NS * C0 + s * C1, C1)

        plsc.subcore_barrier()
        pltpu.sync_copy(deg_sh.at[pl.ds(s * slc, slc)],
                        out_hbm.at[c, pl.ds(s * slc, slc)])

    return deg_k


NBUF = 2


def _make_scatter_kernel(C0, C1):
    cmax = max(C0, C1)
    rps = NPAD // NS
    mesh = plsc.VectorSubcoreMesh(core_axis_name="c", subcore_axis_name="s")

    @functools.partial(
        pl.kernel,
        out_type=jax.ShapeDtypeStruct((NC, NPAD, F), jnp.float32),
        mesh=mesh,
        scratch_types=[
            pltpu.VMEM((NBUF, 1, K), jnp.int32),
            pltpu.VMEM((NBUF, 1, K), jnp.int32),
            pltpu.VMEM((NBUF, K, F), jnp.float32),
            pltpu.VMEM_SHARED((NPAD, F), jnp.float32),
            pltpu.SemaphoreType.DMA((NBUF,)),
            pltpu.SemaphoreType.DMA((NBUF,)),
            pltpu.SemaphoreType.DMA((NBUF,)),
            pltpu.SemaphoreType.DMA,
        ],
    )
    def scat_k(src_hbm, dst_hbm, g_hbm, out_hbm, idx_s, idx_d, rows, acc_sh,
               sem_i, sem_j, sem_g, sem_s):
        c = lax.axis_index("c")
        s = lax.axis_index("s")

        def zbody(i, carry):
            for j in range(F // LANES):
                rows[0, i, pl.ds(j * LANES, LANES)] = jnp.zeros(
                    (LANES,), jnp.float32)
            return carry

        lax.fori_loop(0, K, zbody, 0)
        for t in range(rps // K):
            pltpu.sync_copy(rows.at[0], acc_sh.at[pl.ds(s * rps + t * K, K)])
        plsc.subcore_barrier()

        def run(cbase, nchunks):
            for b in range(NBUF):
                pltpu.async_copy(src_hbm.at[cbase + b], idx_s.at[b],
                                 sem_i.at[b])
                pltpu.async_copy(dst_hbm.at[cbase + b], idx_d.at[b],
                                 sem_j.at[b])

            def body(grp, carry):
                base = grp * NBUF
                gds = []
                for b in range(NBUF):
                    pltpu.make_async_copy(src_hbm.at[cbase + base + b],
                                          idx_s.at[b], sem_i.at[b]).wait()
                    gds.append(
                        pltpu.async_copy(g_hbm.at[idx_s.at[b, 0]], rows.at[b],
                                         sem_g.at[b]))
                sds = []
                for b in range(NBUF):
                    gds[b].wait()
                    nc = base + b + NBUF

                    @pl.when(nc < nchunks)
                    def _():
                        pltpu.async_copy(src_hbm.at[cbase + nc], idx_s.at[b],
                                         sem_i.at[b])

                    pltpu.make_async_copy(dst_hbm.at[cbase + base + b],
                                          idx_d.at[b], sem_j.at[b]).wait()
                    sds.append(
                        pltpu.async_copy(rows.at[b],
                                         acc_sh.at[idx_d.at[b, 0]],
                                         sem_s, add=True))
                for d in sds:
                    d.wait()
                for b in range(NBUF):
                    nc = base + b + NBUF

                    @pl.when(nc < nchunks)
                    def _():
                        pltpu.async_copy(dst_hbm.at[cbase + nc], idx_d.at[b],
                                         sem_j.at[b])

                return carry

            lax.fori_loop(0, nchunks // NBUF, body, 0)

        @pl.when(c == 0)
        def _():
            run(s * C0, C0)

        if C1 > 0:
            @pl.when(c == 1)
            def _():
                run(NS * C0 + s * C1, C1)

        plsc.subcore_barrier()
        for t in range(rps // K):
            pltpu.sync_copy(acc_sh.at[pl.ds(s * rps + t * K, K)],
                            out_hbm.at[c, pl.ds(s * rps + t * K, K)])

    return scat_k


BR = 400


def _tc_dinv(degp):
    def body(degp_ref, dinv_ref):
        dinv_ref[...] = lax.rsqrt(degp_ref[0] + degp_ref[1] + 1.0)

    return pl.pallas_call(
        body,
        in_specs=[pl.BlockSpec((NC, NPAD), lambda: (0, 0))],
        out_specs=pl.BlockSpec((NPAD,), lambda: (0,)),
        out_shape=jax.ShapeDtypeStruct((NPAD,), jnp.float32),
    )(degp).reshape(NPAD, 1)


def _tc_prep(x, W1, dinv):
    def body(x_ref, w_ref, dinv_ref, g_ref):
        h = lax.dot_general(x_ref[...], w_ref[...], (((1,), (0,)), ((), ())),
                            preferred_element_type=jnp.float32)
        g_ref[...] = h * dinv_ref[...]

    return pl.pallas_call(
        body,
        grid=(N // BR,),
        in_specs=[
            pl.BlockSpec((BR, F), lambda i: (i, 0)),
            pl.BlockSpec((F, F), lambda i: (0, 0)),
            pl.BlockSpec((BR, 1), lambda i: (i, 0)),
        ],
        out_specs=pl.BlockSpec((BR, F), lambda i: (i, 0)),
        out_shape=jax.ShapeDtypeStruct((N, F), jnp.float32),
    )(x, W1, dinv)


def _tc_mid(p, g1, dinv, b1, W2):
    def body(p_ref, g_ref, dinv_ref, b_ref, w_ref, h_ref, g2_ref):
        acc = p_ref[0] + p_ref[1] + g_ref[...]
        dv = dinv_ref[...]
        h1 = jnp.maximum(acc * dv + b_ref[...], 0.0)
        h_ref[...] = h1
        g2 = lax.dot_general(h1, w_ref[...], (((1,), (0,)), ((), ())),
                             preferred_element_type=jnp.float32)
        g2_ref[...] = g2 * dv

    return pl.pallas_call(
        body,
        grid=(N // BR,),
        in_specs=[
            pl.BlockSpec((NC, BR, F), lambda i: (0, i, 0)),
            pl.BlockSpec((BR, F), lambda i: (i, 0)),
            pl.BlockSpec((BR, 1), lambda i: (i, 0)),
            pl.BlockSpec((1, F), lambda i: (0, 0)),
            pl.BlockSpec((F, F), lambda i: (0, 0)),
        ],
        out_specs=[
            pl.BlockSpec((BR, F), lambda i: (i, 0)),
            pl.BlockSpec((BR, F), lambda i: (i, 0)),
        ],
        out_shape=[
            jax.ShapeDtypeStruct((N, F), jnp.float32),
            jax.ShapeDtypeStruct((N, F), jnp.float32),
        ],
    )(p, g1, dinv, b1, W2)


def _tc_final(p, g2, dinv, b2, h1, linw, linb):
    ncls = linw.shape[-1]

    def body(p_ref, g_ref, dinv_ref, b_ref, h1_ref, lw_ref, lb_ref, out_ref):
        acc = p_ref[0] + p_ref[1] + g_ref[...]
        h2 = jnp.maximum(acc * dinv_ref[...] + b_ref[...], 0.0)
        logits = (
            lax.dot_general(h1_ref[...], lw_ref[0], (((1,), (0,)), ((), ())),
                            preferred_element_type=jnp.float32)
            + lax.dot_general(h2, lw_ref[1], (((1,), (0,)), ((), ())),
                              preferred_element_type=jnp.float32)
            + lb_ref[...]
        )
        m = jnp.max(logits, axis=1, keepdims=True)
        ex = jnp.exp(logits - m)
        out_ref[...] = (logits - m) - jnp.log(jnp.sum(ex, axis=1, keepdims=True))

    return pl.pallas_call(
        body,
        grid=(N // BR,),
        in_specs=[
            pl.BlockSpec((NC, BR, F), lambda i: (0, i, 0)),
            pl.BlockSpec((BR, F), lambda i: (i, 0)),
            pl.BlockSpec((BR, 1), lambda i: (i, 0)),
            pl.BlockSpec((1, F), lambda i: (0, 0)),
            pl.BlockSpec((BR, F), lambda i: (i, 0)),
            pl.BlockSpec((2, F, ncls), lambda i: (0, 0, 0)),
            pl.BlockSpec((1, ncls), lambda i: (0, 0)),
        ],
        out_specs=pl.BlockSpec((BR, ncls), lambda i: (i, 0)),
        out_shape=jax.ShapeDtypeStruct((N, ncls), jnp.float32),
    )(p, g2, dinv, b2, h1, linw, linb)


def kernel(x, edge_index, W1, b1, W2, b2, lin_W, lin_b):
    E = edge_index.shape[1]
    src = edge_index[0].astype(jnp.int32)
    dst = edge_index[1].astype(jnp.int32)
    C0, C1 = 150, 10
    assert NS * (C0 + C1) * K >= E
    epad = NS * (C0 + C1) * K
    padn = epad - E
    src = jnp.concatenate([src, jnp.zeros((padn,), jnp.int32)]).reshape(-1, 1, K)
    dst = jnp.concatenate(
        [dst, N + (jnp.arange(padn, dtype=jnp.int32) % K)]).reshape(-1, 1, K)

    degp = _make_deg_kernel(C0, C1)(dst)
    dinv = _tc_dinv(degp)
    g1 = _tc_prep(x, W1, dinv)
    scat = _make_scatter_kernel(C0, C1)
    p1 = scat(src, dst, g1)
    h1, g2 = _tc_mid(p1, g1, dinv, b1.reshape(1, F), W2)
    p2 = scat(src, dst, g2)
    out = _tc_final(p2, g2, dinv, b2.reshape(1, F), h1,
                    lin_W.reshape(2, F, -1), lin_b.reshape(1, -1))
    return out
